# baseline (speedup 1.0000x reference)
import functools

import numpy as np
import jax
import jax.numpy as jnp
from jax import lax
from jax.experimental import pallas as pl
from jax.experimental.pallas import tpu as pltpu
from jax.experimental.pallas import tpu_sc as plsc

N = 10000
E = 320000
D = 128
PW = 16
N_G = 10
U_MAX = 3.0
STEP = 0.3
GAMMA = 10.0
LOG2 = float(np.log(2.0))

NC = 2
NS = 16
L = 16
NW = NC * NS
EPW = E // NW
CH = 128
NFULL = EPW // CH
TAIL = EPW - NFULL * CH

K_TAB = 2048
U_CLIP = 23.04
INV_DU = float(K_TAB - 1) / U_CLIP
TB = 2048
CHP = 64
SUP = 512
NSUP = EPW // SUP
PSUB = (EPW - NSUP * SUP - TAIL) // CHP
TROWS = K_TAB // NS


def _ssplus(h):
    return jnp.maximum(h, 0.0) + jnp.log1p(jnp.exp(-jnp.abs(h))) - LOG2


def _lin_body(x_ref, w_ref, b_ref, o_ref):
    o_ref[...] = (
        jnp.dot(x_ref[...], w_ref[...], preferred_element_type=jnp.float32)
        + b_ref[...]
    )


def _tc_lin(x, w, b):
    nb = 5
    rows = N // nb
    return pl.pallas_call(
        _lin_body,
        grid=(nb,),
        in_specs=[
            pl.BlockSpec((rows, D), lambda i: (i, 0)),
            pl.BlockSpec((D, D), lambda i: (0, 0)),
            pl.BlockSpec((1, D), lambda i: (0, 0)),
        ],
        out_specs=pl.BlockSpec((rows, D), lambda i: (i, 0)),
        out_shape=jax.ShapeDtypeStruct((N, D), jnp.float32),
    )(x, w, b.reshape(1, D))


def _table_body(f1_ref, b1_ref, f2_ref, b2_ref, o_ref):
    i = pl.program_id(0)
    rows = lax.broadcasted_iota(jnp.int32, (TB, 1), 0) + i * TB
    u = rows.astype(jnp.float32) * (U_CLIP / (K_TAB - 1))
    d = jnp.sqrt(u)
    offs = lax.broadcasted_iota(jnp.int32, (1, PW), 1).astype(jnp.float32) * STEP
    rbf = jnp.exp(-GAMMA * (d - offs) ** 2)
    h = jnp.dot(rbf, f1_ref[...], preferred_element_type=jnp.float32) + b1_ref[...]
    h = _ssplus(h)
    h = jnp.dot(h, f2_ref[...], preferred_element_type=jnp.float32) + b2_ref[...]
    o_ref[...] = _ssplus(h)


def _tc_table(f1p, f1_b, f2_W, f2_b):
    nb = K_TAB // TB
    return pl.pallas_call(
        _table_body,
        grid=(nb,),
        in_specs=[
            pl.BlockSpec((PW, D), lambda i: (0, 0)),
            pl.BlockSpec((1, D), lambda i: (0, 0)),
            pl.BlockSpec((D, D), lambda i: (0, 0)),
            pl.BlockSpec((1, D), lambda i: (0, 0)),
        ],
        out_specs=pl.BlockSpec((TB, D), lambda i: (i, 0)),
        out_shape=jax.ShapeDtypeStruct((K_TAB, D), jnp.float32),
    )(f1p, f1_b.reshape(1, D), f2_W, f2_b.reshape(1, D))


SUPA = 512
NSA = EPW // SUPA
REMG = (EPW - NSA * SUPA) // L


@functools.lru_cache(maxsize=None)
def _sc_kidx():
    mesh = plsc.VectorSubcoreMesh(core_axis_name="c", subcore_axis_name="s")

    @functools.partial(
        pl.kernel,
        out_type=jax.ShapeDtypeStruct((E,), jnp.int32),
        mesh=mesh,
        scratch_types=[
            pltpu.VMEM((N,), jnp.float32),
            pltpu.VMEM((N,), jnp.float32),
            pltpu.VMEM((N,), jnp.float32),
            pltpu.VMEM((SUPA,), jnp.int32),
            pltpu.VMEM((SUPA,), jnp.int32),
            pltpu.VMEM((SUPA,), jnp.int32),
        ],
        compiler_params=pltpu.CompilerParams(needs_layout_passes=False),
    )
    def k(px, py, pz, src, dst, kidx_out, pxv, pyv, pzv, sidx, didx, kbuf):
        cid = lax.axis_index("c")
        sid = lax.axis_index("s")
        wid = sid * NC + cid
        base = wid * EPW

        pltpu.sync_copy(px, pxv)
        pltpu.sync_copy(py, pyv)
        pltpu.sync_copy(pz, pzv)

        def chunk(off, ng):
            off = pl.multiple_of(off, 16)
            ne = ng * L
            pltpu.sync_copy(src.at[pl.ds(off, ne)], sidx.at[pl.ds(0, ne)])
            pltpu.sync_copy(dst.at[pl.ds(off, ne)], didx.at[pl.ds(0, ne)])
            for g in range(ng):
                si = sidx[pl.ds(g * L, L)]
                di = didx[pl.ds(g * L, L)]
                dx = plsc.load_gather(pxv, [si]) - plsc.load_gather(pxv, [di])
                dy = plsc.load_gather(pyv, [si]) - plsc.load_gather(pyv, [di])
                dz = plsc.load_gather(pzv, [si]) - plsc.load_gather(pzv, [di])
                u = dx * dx + dy * dy + dz * dz
                kf = jnp.minimum(u * INV_DU + 0.5, float(K_TAB - 1))
                kbuf[pl.ds(g * L, L)] = kf.astype(jnp.int32)
            pltpu.sync_copy(kbuf.at[pl.ds(0, ne)], kidx_out.at[pl.ds(off, ne)])

        def body(i, carry):
            chunk(base + i * SUPA, SUPA // L)
            return carry

        lax.fori_loop(0, NSA, body, 0)
        chunk(base + NSA * SUPA, REMG)

    return k


@functools.lru_cache(maxsize=None)
def _sc_cfconv():
    mesh = plsc.VectorSubcoreMesh(core_axis_name="c", subcore_axis_name="s")

    @functools.partial(
        pl.kernel,
        out_type=jax.ShapeDtypeStruct((NC, N, D), jnp.float32),
        mesh=mesh,
        scratch_types=[
            pltpu.VMEM((SUP,), jnp.int32),
            pltpu.VMEM((SUP,), jnp.int32),
            pltpu.VMEM((SUP,), jnp.int32),
            pltpu.VMEM((CHP,), jnp.int32),
            pltpu.VMEM((CHP,), jnp.int32),
            pltpu.VMEM((CHP, D), jnp.float32),
            pltpu.VMEM((CHP, D), jnp.float32),
            pltpu.VMEM((CHP, D), jnp.float32),
            pltpu.VMEM((CHP, D), jnp.float32),
            pltpu.VMEM_SHARED((N, D), jnp.float32),
            pltpu.VMEM_SHARED((K_TAB, D), jnp.float32),
            pltpu.SemaphoreType.DMA,
            pltpu.SemaphoreType.DMA,
            pltpu.SemaphoreType.DMA,
            pltpu.SemaphoreType.DMA,
            pltpu.SemaphoreType.DMA,
            pltpu.SemaphoreType.DMA,
        ],
    )
    def k(m, tab, src, dst, kidx, zeros, v_out,
          sidx_big, didx_big, kbuf_big, didx0, didx1,
          mrow0, mrow1, wrow0, wrow1, acc, tabs,
          semm0, semm1, semw0, semw1, semsc0, semsc1):
        cid = lax.axis_index("c")
        sid = lax.axis_index("s")
        wid = sid * NC + cid
        base = wid * EPW

        mrows = (mrow0, mrow1)
        wrows = (wrow0, wrow1)
        didxs = (didx0, didx1)
        semms = (semm0, semm1)
        semws = (semw0, semw1)
        semscs = (semsc0, semsc1)

        rstart = pl.multiple_of(jnp.minimum(sid * 640, N - 640), 8)
        pltpu.sync_copy(zeros.at[pl.ds(rstart, 640)],
                        acc.at[pl.ds(rstart, 640)])
        tstart = pl.multiple_of(sid * TROWS, 8)
        pltpu.sync_copy(tab.at[pl.ds(tstart, TROWS)],
                        tabs.at[pl.ds(tstart, TROWS)])
        plsc.subcore_barrier()

        def run_super(soff, nsub):
            ne = nsub * CHP
            pltpu.sync_copy(src.at[pl.ds(soff, ne)], sidx_big.at[pl.ds(0, ne)])
            pltpu.sync_copy(dst.at[pl.ds(soff, ne)], didx_big.at[pl.ds(0, ne)])
            pltpu.sync_copy(kidx.at[pl.ds(soff, ne)], kbuf_big.at[pl.ds(0, ne)])

            mcopies = {}
            wcopies = {}
            scatters = {}

            def issue(c):
                b = c & 1
                if c >= 2:
                    scatters.pop(c - 2).wait()
                mcopies[c] = pltpu.async_copy(
                    m.at[sidx_big.at[pl.ds(c * CHP, CHP)]], mrows[b], semms[b])
                wcopies[c] = pltpu.async_copy(
                    tabs.at[kbuf_big.at[pl.ds(c * CHP, CHP)]], wrows[b],
                    semws[b])

            def compute(c):
                b = c & 1
                mcopies.pop(c).wait()
                wcopies.pop(c).wait()
                wrow = wrows[b]
                mr = mrows[b]

                def mbody(e, carry):
                    for l in range(D // L):
                        s = pl.ds(l * L, L)
                        mr[e, s] = mr[e, s] * wrow[e, s]
                    return carry

                lax.fori_loop(0, CHP, mbody, 0)
                di = didxs[b]
                for l in range(CHP // L):
                    di[pl.ds(l * L, L)] = didx_big[pl.ds(c * CHP + l * L, L)]
                scatters[c] = pltpu.async_copy(
                    mr, acc.at[di], semscs[b], add=True)

            issue(0)
            for c in range(nsub):
                if c + 1 < nsub:
                    issue(c + 1)
                compute(c)
            for c in sorted(scatters):
                scatters.pop(c).wait()

        def body(s, carry):
            run_super(pl.multiple_of(base + s * SUP, 16), SUP // CHP)
            return carry

        lax.fori_loop(0, NSUP, body, 0)
        run_super(pl.multiple_of(base + NSUP * SUP, 16), PSUB)

        toff = pl.multiple_of(base + NSUP * SUP + PSUB * CHP, 16)
        pltpu.sync_copy(src.at[pl.ds(toff, TAIL)], sidx_big.at[pl.ds(0, TAIL)])
        pltpu.sync_copy(dst.at[pl.ds(toff, TAIL)], didx0.at[pl.ds(0, TAIL)])
        pltpu.sync_copy(kidx.at[pl.ds(toff, TAIL)], kbuf_big.at[pl.ds(0, TAIL)])
        cm = pltpu.async_copy(
            m.at[sidx_big.at[pl.ds(0, TAIL)]], mrow0.at[pl.ds(0, TAIL)], semm0)
        cw = pltpu.async_copy(
            tabs.at[kbuf_big.at[pl.ds(0, TAIL)]], wrow0.at[pl.ds(0, TAIL)],
            semw0)
        cm.wait()
        cw.wait()

        def tbody(e, carry):
            for l in range(D // L):
                s = pl.ds(l * L, L)
                mrow0[e, s] = mrow0[e, s] * wrow0[e, s]
            return carry

        lax.fori_loop(0, TAIL, tbody, 0)
        pltpu.sync_copy(mrow0.at[pl.ds(0, TAIL)],
                        acc.at[didx0.at[pl.ds(0, TAIL)]], add=True)

        plsc.subcore_barrier()
        pltpu.sync_copy(acc.at[pl.ds(rstart, 640)],
                        v_out.at[cid, pl.ds(rstart, 640)])

    return k


def _final_body(x_ref, v_ref, w1_ref, b1_ref, w2_ref, b2_ref, o_ref):
    v = v_ref[0] + v_ref[1]
    h = _ssplus(
        jnp.dot(v, w1_ref[...], preferred_element_type=jnp.float32) + b1_ref[...])
    o_ref[...] = (
        x_ref[...]
        + jnp.dot(h, w2_ref[...], preferred_element_type=jnp.float32)
        + b2_ref[...]
    )


def _tc_final(x, vpart, w1, b1, w2, b2):
    nb = 5
    rows = N // nb
    return pl.pallas_call(
        _final_body,
        grid=(nb,),
        in_specs=[
            pl.BlockSpec((rows, D), lambda i: (i, 0)),
            pl.BlockSpec((NC, rows, D), lambda i: (0, i, 0)),
            pl.BlockSpec((D, D), lambda i: (0, 0)),
            pl.BlockSpec((1, D), lambda i: (0, 0)),
            pl.BlockSpec((D, D), lambda i: (0, 0)),
            pl.BlockSpec((1, D), lambda i: (0, 0)),
        ],
        out_specs=pl.BlockSpec((rows, D), lambda i: (i, 0)),
        out_shape=jax.ShapeDtypeStruct((N, D), jnp.float32),
    )(x, vpart, w1, b1.reshape(1, D), w2, b2.reshape(1, D))


def kernel(x, edge_index, z, position, lin1_W, lin1_b, f1_W, f1_b, f2_W, f2_b,
           mlp1_W, mlp1_b, mlp2_W, mlp2_b):
    src = edge_index[0]
    dst = edge_index[1]
    px = position[:, 0]
    py = position[:, 1]
    pz = position[:, 2]
    f1p = jnp.pad(f1_W, ((0, PW - N_G), (0, 0)))
    zeros = jnp.zeros((N, D), jnp.float32)

    m = _tc_lin(x, lin1_W, lin1_b)
    tab = _tc_table(f1p, f1_b, f2_W, f2_b)
    kidx = _sc_kidx()(px, py, pz, src, dst)
    vpart = _sc_cfconv()(m, tab, src, dst, kidx, zeros)
    return _tc_final(x, vpart, mlp1_W, mlp1_b, mlp2_W, mlp2_b)


# speedup vs baseline: 1.0242x; 1.0242x over previous
import functools

import numpy as np
import jax
import jax.numpy as jnp
from jax import lax
from jax.experimental import pallas as pl
from jax.experimental.pallas import tpu as pltpu
from jax.experimental.pallas import tpu_sc as plsc

N = 10000
E = 320000
D = 128
PW = 16
N_G = 10
U_MAX = 3.0
STEP = 0.3
GAMMA = 10.0
LOG2 = float(np.log(2.0))

NC = 2
NS = 16
L = 16
NW = NC * NS
EPW = E // NW
CH = 128
NFULL = EPW // CH
TAIL = EPW - NFULL * CH

K_TAB = 2048
U_CLIP = 23.04
INV_DU = float(K_TAB - 1) / U_CLIP
TB = 2048
CHP = 64
SUP = 512
NSUP = EPW // SUP
PSUB = (EPW - NSUP * SUP - TAIL) // CHP
TROWS = K_TAB // NS


def _ssplus(h):
    return jnp.maximum(h, 0.0) + jnp.log1p(jnp.exp(-jnp.abs(h))) - LOG2


def _lin_body(x_ref, w_ref, b_ref, o_ref):
    o_ref[...] = (
        jnp.dot(x_ref[...], w_ref[...], preferred_element_type=jnp.float32)
        + b_ref[...]
    )


def _tc_lin(x, w, b):
    nb = 5
    rows = N // nb
    return pl.pallas_call(
        _lin_body,
        grid=(nb,),
        in_specs=[
            pl.BlockSpec((rows, D), lambda i: (i, 0)),
            pl.BlockSpec((D, D), lambda i: (0, 0)),
            pl.BlockSpec((1, D), lambda i: (0, 0)),
        ],
        out_specs=pl.BlockSpec((rows, D), lambda i: (i, 0)),
        out_shape=jax.ShapeDtypeStruct((N, D), jnp.float32),
    )(x, w, b.reshape(1, D))


def _table_body(f1_ref, b1_ref, f2_ref, b2_ref, o_ref):
    i = pl.program_id(0)
    rows = lax.broadcasted_iota(jnp.int32, (TB, 1), 0) + i * TB
    u = rows.astype(jnp.float32) * (U_CLIP / (K_TAB - 1))
    d = jnp.sqrt(u)
    offs = lax.broadcasted_iota(jnp.int32, (1, PW), 1).astype(jnp.float32) * STEP
    rbf = jnp.exp(-GAMMA * (d - offs) ** 2)
    h = jnp.dot(rbf, f1_ref[...], preferred_element_type=jnp.float32) + b1_ref[...]
    h = _ssplus(h)
    h = jnp.dot(h, f2_ref[...], preferred_element_type=jnp.float32) + b2_ref[...]
    o_ref[...] = _ssplus(h)


def _tc_table(f1p, f1_b, f2_W, f2_b):
    nb = K_TAB // TB
    return pl.pallas_call(
        _table_body,
        grid=(nb,),
        in_specs=[
            pl.BlockSpec((PW, D), lambda i: (0, 0)),
            pl.BlockSpec((1, D), lambda i: (0, 0)),
            pl.BlockSpec((D, D), lambda i: (0, 0)),
            pl.BlockSpec((1, D), lambda i: (0, 0)),
        ],
        out_specs=pl.BlockSpec((TB, D), lambda i: (i, 0)),
        out_shape=jax.ShapeDtypeStruct((K_TAB, D), jnp.float32),
    )(f1p, f1_b.reshape(1, D), f2_W, f2_b.reshape(1, D))


SUPA = 1024
NSA = EPW // SUPA
REMG = (EPW - NSA * SUPA) // L


@functools.lru_cache(maxsize=None)
def _sc_kidx():
    mesh = plsc.VectorSubcoreMesh(core_axis_name="c", subcore_axis_name="s")

    @functools.partial(
        pl.kernel,
        out_type=jax.ShapeDtypeStruct((E,), jnp.int32),
        mesh=mesh,
        scratch_types=[
            pltpu.VMEM((N,), jnp.float32),
            pltpu.VMEM((N,), jnp.float32),
            pltpu.VMEM((N,), jnp.float32),
            pltpu.VMEM((SUPA,), jnp.int32),
            pltpu.VMEM((SUPA,), jnp.int32),
            pltpu.VMEM((SUPA,), jnp.int32),
        ],
        compiler_params=pltpu.CompilerParams(needs_layout_passes=False),
    )
    def k(px, py, pz, src, dst, kidx_out, pxv, pyv, pzv, sidx, didx, kbuf):
        cid = lax.axis_index("c")
        sid = lax.axis_index("s")
        wid = sid * NC + cid
        base = wid * EPW

        pltpu.sync_copy(px, pxv)
        pltpu.sync_copy(py, pyv)
        pltpu.sync_copy(pz, pzv)

        def chunk(off, ng):
            off = pl.multiple_of(off, 16)
            ne = ng * L
            pltpu.sync_copy(src.at[pl.ds(off, ne)], sidx.at[pl.ds(0, ne)])
            pltpu.sync_copy(dst.at[pl.ds(off, ne)], didx.at[pl.ds(0, ne)])
            for g in range(ng):
                si = sidx[pl.ds(g * L, L)]
                di = didx[pl.ds(g * L, L)]
                dx = plsc.load_gather(pxv, [si]) - plsc.load_gather(pxv, [di])
                dy = plsc.load_gather(pyv, [si]) - plsc.load_gather(pyv, [di])
                dz = plsc.load_gather(pzv, [si]) - plsc.load_gather(pzv, [di])
                u = dx * dx + dy * dy + dz * dz
                kf = jnp.minimum(u * INV_DU + 0.5, float(K_TAB - 1))
                kbuf[pl.ds(g * L, L)] = kf.astype(jnp.int32)
            pltpu.sync_copy(kbuf.at[pl.ds(0, ne)], kidx_out.at[pl.ds(off, ne)])

        def body(i, carry):
            chunk(base + i * SUPA, SUPA // L)
            return carry

        lax.fori_loop(0, NSA, body, 0)
        chunk(base + NSA * SUPA, REMG)

    return k


@functools.lru_cache(maxsize=None)
def _sc_cfconv():
    mesh = plsc.VectorSubcoreMesh(core_axis_name="c", subcore_axis_name="s")

    @functools.partial(
        pl.kernel,
        out_type=jax.ShapeDtypeStruct((NC, N, D), jnp.float32),
        mesh=mesh,
        scratch_types=[
            pltpu.VMEM((SUP,), jnp.int32),
            pltpu.VMEM((SUP,), jnp.int32),
            pltpu.VMEM((SUP,), jnp.int32),
            pltpu.VMEM((CHP,), jnp.int32),
            pltpu.VMEM((CHP,), jnp.int32),
            pltpu.VMEM((CHP, D), jnp.float32),
            pltpu.VMEM((CHP, D), jnp.float32),
            pltpu.VMEM((CHP, D), jnp.float32),
            pltpu.VMEM((CHP, D), jnp.float32),
            pltpu.VMEM_SHARED((N, D), jnp.float32),
            pltpu.VMEM_SHARED((K_TAB, D), jnp.float32),
            pltpu.SemaphoreType.DMA,
            pltpu.SemaphoreType.DMA,
            pltpu.SemaphoreType.DMA,
            pltpu.SemaphoreType.DMA,
            pltpu.SemaphoreType.DMA,
            pltpu.SemaphoreType.DMA,
        ],
    )
    def k(m, tab, src, dst, kidx, zeros, v_out,
          sidx_big, didx_big, kbuf_big, didx0, didx1,
          mrow0, mrow1, wrow0, wrow1, acc, tabs,
          semm0, semm1, semw0, semw1, semsc0, semsc1):
        cid = lax.axis_index("c")
        sid = lax.axis_index("s")
        wid = sid * NC + cid
        base = wid * EPW

        mrows = (mrow0, mrow1)
        wrows = (wrow0, wrow1)
        didxs = (didx0, didx1)
        semms = (semm0, semm1)
        semws = (semw0, semw1)
        semscs = (semsc0, semsc1)

        rstart = pl.multiple_of(jnp.minimum(sid * 640, N - 640), 8)
        pltpu.sync_copy(zeros.at[pl.ds(rstart, 640)],
                        acc.at[pl.ds(rstart, 640)])
        tstart = pl.multiple_of(sid * TROWS, 8)
        pltpu.sync_copy(tab.at[pl.ds(tstart, TROWS)],
                        tabs.at[pl.ds(tstart, TROWS)])
        plsc.subcore_barrier()

        def run_super(soff, nsub):
            ne = nsub * CHP
            pltpu.sync_copy(src.at[pl.ds(soff, ne)], sidx_big.at[pl.ds(0, ne)])
            pltpu.sync_copy(dst.at[pl.ds(soff, ne)], didx_big.at[pl.ds(0, ne)])
            pltpu.sync_copy(kidx.at[pl.ds(soff, ne)], kbuf_big.at[pl.ds(0, ne)])

            mcopies = {}
            wcopies = {}
            scatters = {}

            def issue(c):
                b = c & 1
                if c >= 2:
                    scatters.pop(c - 2).wait()
                mcopies[c] = pltpu.async_copy(
                    m.at[sidx_big.at[pl.ds(c * CHP, CHP)]], mrows[b], semms[b])
                wcopies[c] = pltpu.async_copy(
                    tabs.at[kbuf_big.at[pl.ds(c * CHP, CHP)]], wrows[b],
                    semws[b])

            def compute(c):
                b = c & 1
                mcopies.pop(c).wait()
                wcopies.pop(c).wait()
                wrow = wrows[b]
                mr = mrows[b]

                def mbody(e, carry):
                    for l in range(D // L):
                        s = pl.ds(l * L, L)
                        mr[e, s] = mr[e, s] * wrow[e, s]
                    return carry

                lax.fori_loop(0, CHP, mbody, 0)
                di = didxs[b]
                for l in range(CHP // L):
                    di[pl.ds(l * L, L)] = didx_big[pl.ds(c * CHP + l * L, L)]
                scatters[c] = pltpu.async_copy(
                    mr, acc.at[di], semscs[b], add=True)

            issue(0)
            for c in range(nsub):
                if c + 1 < nsub:
                    issue(c + 1)
                compute(c)
            for c in sorted(scatters):
                scatters.pop(c).wait()

        def body(s, carry):
            run_super(pl.multiple_of(base + s * SUP, 16), SUP // CHP)
            return carry

        lax.fori_loop(0, NSUP, body, 0)
        run_super(pl.multiple_of(base + NSUP * SUP, 16), PSUB)

        toff = pl.multiple_of(base + NSUP * SUP + PSUB * CHP, 16)
        pltpu.sync_copy(src.at[pl.ds(toff, TAIL)], sidx_big.at[pl.ds(0, TAIL)])
        pltpu.sync_copy(dst.at[pl.ds(toff, TAIL)], didx0.at[pl.ds(0, TAIL)])
        pltpu.sync_copy(kidx.at[pl.ds(toff, TAIL)], kbuf_big.at[pl.ds(0, TAIL)])
        cm = pltpu.async_copy(
            m.at[sidx_big.at[pl.ds(0, TAIL)]], mrow0.at[pl.ds(0, TAIL)], semm0)
        cw = pltpu.async_copy(
            tabs.at[kbuf_big.at[pl.ds(0, TAIL)]], wrow0.at[pl.ds(0, TAIL)],
            semw0)
        cm.wait()
        cw.wait()

        def tbody(e, carry):
            for l in range(D // L):
                s = pl.ds(l * L, L)
                mrow0[e, s] = mrow0[e, s] * wrow0[e, s]
            return carry

        lax.fori_loop(0, TAIL, tbody, 0)
        pltpu.sync_copy(mrow0.at[pl.ds(0, TAIL)],
                        acc.at[didx0.at[pl.ds(0, TAIL)]], add=True)

        plsc.subcore_barrier()
        pltpu.sync_copy(acc.at[pl.ds(rstart, 640)],
                        v_out.at[cid, pl.ds(rstart, 640)])

    return k


def _final_body(x_ref, v_ref, w1_ref, b1_ref, w2_ref, b2_ref, o_ref):
    v = v_ref[0] + v_ref[1]
    h = _ssplus(
        jnp.dot(v, w1_ref[...], preferred_element_type=jnp.float32) + b1_ref[...])
    o_ref[...] = (
        x_ref[...]
        + jnp.dot(h, w2_ref[...], preferred_element_type=jnp.float32)
        + b2_ref[...]
    )


def _tc_final(x, vpart, w1, b1, w2, b2):
    nb = 5
    rows = N // nb
    return pl.pallas_call(
        _final_body,
        grid=(nb,),
        in_specs=[
            pl.BlockSpec((rows, D), lambda i: (i, 0)),
            pl.BlockSpec((NC, rows, D), lambda i: (0, i, 0)),
            pl.BlockSpec((D, D), lambda i: (0, 0)),
            pl.BlockSpec((1, D), lambda i: (0, 0)),
            pl.BlockSpec((D, D), lambda i: (0, 0)),
            pl.BlockSpec((1, D), lambda i: (0, 0)),
        ],
        out_specs=pl.BlockSpec((rows, D), lambda i: (i, 0)),
        out_shape=jax.ShapeDtypeStruct((N, D), jnp.float32),
    )(x, vpart, w1, b1.reshape(1, D), w2, b2.reshape(1, D))


def kernel(x, edge_index, z, position, lin1_W, lin1_b, f1_W, f1_b, f2_W, f2_b,
           mlp1_W, mlp1_b, mlp2_W, mlp2_b):
    src = edge_index[0]
    dst = edge_index[1]
    px = position[:, 0]
    py = position[:, 1]
    pz = position[:, 2]
    f1p = jnp.pad(f1_W, ((0, PW - N_G), (0, 0)))
    zeros = jnp.zeros((N, D), jnp.float32)

    m = _tc_lin(x, lin1_W, lin1_b)
    tab = _tc_table(f1p, f1_b, f2_W, f2_b)
    kidx = _sc_kidx()(px, py, pz, src, dst)
    vpart = _sc_cfconv()(m, tab, src, dst, kidx, zeros)
    return _tc_final(x, vpart, mlp1_W, mlp1_b, mlp2_W, mlp2_b)
